# baseline (speedup 1.0000x reference)
import functools

import jax
import jax.numpy as jnp
from jax import lax
from jax.experimental import pallas as pl
from jax.experimental.pallas import tpu as pltpu
from jax.experimental.pallas import tpu_sc as plsc

N = 10000
E = 320000
D = 128
DE = 16
NG = 64

NC = 2
NS = 16
NW = NC * NS
EW = E // NW
C = 80
NCHUNK = EW // C
NP = 10240
RT = NP // NS


def _sc_mesh():
  return plsc.VectorSubcoreMesh(
      core_axis_name="c", subcore_axis_name="s",
      num_cores=NC, num_subcores=NS)


def _sc_segsum_kernel():
  CH = C // 2

  def body(h_hbm, src_hbm, dst_hbm, z_d_hbm, s_out,
           src_v, dst_v, rows0_v, rows1_v, s_sh, gsem, ssem0, ssem1):
    cid = lax.axis_index("c")
    sid = lax.axis_index("s")
    e0 = pl.multiple_of((cid * NS + sid) * EW, 8)
    r0 = pl.multiple_of(sid * RT, 8)

    pltpu.sync_copy(src_hbm.at[pl.ds(e0, EW)], src_v)
    pltpu.sync_copy(dst_hbm.at[pl.ds(e0, EW)], dst_v)
    pltpu.sync_copy(z_d_hbm, s_sh.at[pl.ds(r0, RT)])
    plsc.subcore_barrier()

    def gather_start(i, rows):
      pltpu.async_copy(
          h_hbm.at[src_v.at[pl.ds(i * C, CH)]], rows.at[pl.ds(0, CH)], gsem)
      pltpu.async_copy(
          h_hbm.at[src_v.at[pl.ds(i * C + CH, CH)]],
          rows.at[pl.ds(CH, CH)], gsem)

    def gather_wait(rows):
      pltpu.make_async_copy(h_hbm.at[pl.ds(0, C)], rows, gsem).wait()

    def scatter_start(i, rows, sem):
      pltpu.async_copy(rows, s_sh.at[dst_v.at[pl.ds(i * C, C)]], sem,
                       add=True)

    def scatter_wait(rows, sem):
      pltpu.make_async_copy(h_hbm.at[pl.ds(0, C)], rows, sem).wait()

    gather_start(0, rows0_v)
    gather_wait(rows0_v)
    scatter_start(0, rows0_v, ssem0)
    gather_start(1, rows1_v)
    gather_wait(rows1_v)
    scatter_start(1, rows1_v, ssem1)
    scatter_wait(rows0_v, ssem0)
    gather_start(2, rows0_v)

    def pair(j, carry):
      a = 2 * j
      gather_wait(rows0_v)
      scatter_start(a, rows0_v, ssem0)
      scatter_wait(rows1_v, ssem1)
      gather_start(a + 1, rows1_v)
      gather_wait(rows1_v)
      scatter_start(a + 1, rows1_v, ssem1)
      scatter_wait(rows0_v, ssem0)
      gather_start(a + 2, rows0_v)
      return carry

    lax.fori_loop(1, (NCHUNK - 1) // 2, pair, 0)
    gather_wait(rows0_v)
    scatter_start(NCHUNK - 1, rows0_v, ssem0)
    scatter_wait(rows1_v, ssem1)
    scatter_wait(rows0_v, ssem0)
    plsc.subcore_barrier()

    pltpu.sync_copy(s_sh.at[pl.ds(r0, RT)], s_out.at[pl.ds(cid * NP + r0, RT)])

  return pl.kernel(
      body,
      out_type=[jax.ShapeDtypeStruct((NC * NP, D), jnp.float32)],
      mesh=_sc_mesh(),
      scratch_types=[
          pltpu.VMEM((EW,), jnp.int32),
          pltpu.VMEM((EW,), jnp.int32),
          pltpu.VMEM((C, D), jnp.float32),
          pltpu.VMEM((C, D), jnp.float32),
          pltpu.VMEM_SHARED((NP, D), jnp.float32),
          pltpu.SemaphoreType.DMA,
          pltpu.SemaphoreType.DMA,
          pltpu.SemaphoreType.DMA,
      ])


def _sc_attr_kernel():
  def body(attrf_hbm, dst_hbm, z_d_hbm, wide_hbm, out,
           dst_v, flat0_v, flat1_v, wide0_v, wide1_v, acc_sh,
           gsem, ssem0, ssem1):
    cid = lax.axis_index("c")
    sid = lax.axis_index("s")
    e0 = pl.multiple_of((cid * NS + sid) * EW, 8)
    r0 = pl.multiple_of(sid * RT, 8)

    pltpu.sync_copy(dst_hbm.at[pl.ds(e0, EW)], dst_v)
    pltpu.sync_copy(z_d_hbm, acc_sh.at[pl.ds(r0, RT)])
    pltpu.sync_copy(wide_hbm, wide0_v)
    pltpu.sync_copy(wide_hbm, wide1_v)
    plsc.subcore_barrier()

    def load_start(i, flat):
      pltpu.async_copy(
          attrf_hbm.at[pl.ds((e0 + i * C) * DE, C * DE)], flat, gsem)

    def load_wait(flat):
      pltpu.make_async_copy(attrf_hbm.at[pl.ds(0, C * DE)], flat, gsem).wait()

    def fill(flat, wide):
      for r in range(C):
        wide[r, 0:DE] = flat[pl.ds(r * DE, DE)]

    def scatter_start(i, wide, sem):
      pltpu.async_copy(wide, acc_sh.at[dst_v.at[pl.ds(i * C, C)]], sem,
                       add=True)

    def scatter_wait(wide, sem):
      pltpu.make_async_copy(z_d_hbm.at[pl.ds(0, C)], wide, sem).wait()

    load_start(0, flat0_v)
    load_wait(flat0_v)
    load_start(1, flat1_v)
    fill(flat0_v, wide0_v)
    scatter_start(0, wide0_v, ssem0)
    load_wait(flat1_v)
    load_start(2, flat0_v)
    fill(flat1_v, wide1_v)
    scatter_start(1, wide1_v, ssem1)

    def pair(j, carry):
      a = 2 * j
      load_wait(flat0_v)
      load_start(a + 1, flat1_v)
      scatter_wait(wide0_v, ssem0)
      fill(flat0_v, wide0_v)
      scatter_start(a, wide0_v, ssem0)
      load_wait(flat1_v)
      load_start(a + 2, flat0_v)
      scatter_wait(wide1_v, ssem1)
      fill(flat1_v, wide1_v)
      scatter_start(a + 1, wide1_v, ssem1)
      return carry

    lax.fori_loop(1, (NCHUNK - 1) // 2, pair, 0)
    load_wait(flat0_v)
    scatter_wait(wide0_v, ssem0)
    fill(flat0_v, wide0_v)
    scatter_start(NCHUNK - 1, wide0_v, ssem0)
    scatter_wait(wide1_v, ssem1)
    scatter_wait(wide0_v, ssem0)
    plsc.subcore_barrier()

    pltpu.sync_copy(acc_sh.at[pl.ds(r0, RT)], out.at[pl.ds(cid * NP + r0, RT)])

  return pl.kernel(
      body,
      out_type=[jax.ShapeDtypeStruct((NC * NP, D), jnp.float32)],
      mesh=_sc_mesh(),
      scratch_types=[
          pltpu.VMEM((EW,), jnp.int32),
          pltpu.VMEM((C * DE,), jnp.float32),
          pltpu.VMEM((C * DE,), jnp.float32),
          pltpu.VMEM((C, D), jnp.float32),
          pltpu.VMEM((C, D), jnp.float32),
          pltpu.VMEM_SHARED((NP, D), jnp.float32),
          pltpu.SemaphoreType.DMA,
          pltpu.SemaphoreType.DMA,
          pltpu.SemaphoreType.DMA,
      ])


BLK = 1024
GRID = NP // BLK


def _dot_t(x, w):
  return lax.dot_general(x, w, (((1,), (1,)), ((), ())),
                         preferred_element_type=jnp.float32,
                         precision=lax.Precision.HIGHEST)


def _embed_body(x_ref, w_ref, b_ref, out_ref):
  out_ref[...] = _dot_t(x_ref[...], w_ref[...]) + b_ref[...]


def _mlp_body(sp0_ref, sp1_ref, h_ref, acc0_ref, acc1_ref,
              we_ref, be_ref, bemat_ref, w1_ref, b1_ref, w2_ref, b2_ref,
              out_ref):
  acc = acc0_ref[...] + acc1_ref[...]
  agg = (sp0_ref[...] + sp1_ref[...] + h_ref[...]
         + _dot_t(acc[:, 0:DE], we_ref[...])
         + _dot_t(acc[:, DE:2 * DE], bemat_ref[...])
         + be_ref[...])
  hid = jnp.maximum(_dot_t(agg, w1_ref[...]) + b1_ref[...], 0.0)
  out_ref[...] = _dot_t(hid, w2_ref[...]) + b2_ref[...]


def _bn_body(last, x_ref, g_ref, bt_ref, batch_ref, wp_ref, bp_ref, out_ref):
  x = x_ref[0:N, :]
  mu = jnp.mean(x, axis=0, keepdims=True)
  ctr = x - mu
  var = jnp.mean(ctr * ctr, axis=0, keepdims=True)
  hn = ctr * lax.rsqrt(var + 1e-5) * g_ref[...] + bt_ref[...]
  if not last:
    out_ref[0:N, :] = jnp.maximum(hn, 0.0)
    out_ref[N:NP, :] = jnp.zeros((NP - N, D), jnp.float32)
  else:
    onehot = (batch_ref[...] ==
              lax.broadcasted_iota(jnp.int32, (N, NG), 1)).astype(jnp.float32)
    cnt = jnp.sum(onehot, axis=0, keepdims=True)
    ohs = onehot / jnp.maximum(cnt, 1.0)
    gmean = lax.dot_general(ohs, hn, (((0,), (0,)), ((), ())),
                            preferred_element_type=jnp.float32,
                            precision=lax.Precision.HIGHEST)
    out_ref[...] = _dot_t(gmean, wp_ref[...]) + bp_ref[...]


def kernel(x, edge_index, edge_attr, batch, W0, b0, We0, be0, W10, b10, W20,
           b20, g0, bt0, We1, be1, W11, b11, W21, b21, g1, bt1, Wp, bp):
  assert x.shape == (N, D) and edge_index.shape == (2, E)

  src = edge_index[0]
  dst = edge_index[1]
  z_d = jnp.zeros((RT, D), jnp.float32)
  wide = jnp.zeros((C, D), jnp.float32).at[:, DE:2 * DE].set(1.0)
  batch2 = jnp.broadcast_to(batch[:, None], (N, NG))
  wp_pad = jnp.zeros((D, D), jnp.float32).at[:1, :].set(Wp)
  bp_pad = jnp.zeros((1, D), jnp.float32).at[:, :1].set(bp[None, :])
  x_pad = jnp.concatenate([x, jnp.zeros((NP - N, D), jnp.float32)], 0)

  row_d = pl.BlockSpec((BLK, D), lambda i: (i, 0))
  row_d2 = pl.BlockSpec((BLK, D), lambda i: (GRID + i, 0))
  row_e = pl.BlockSpec((BLK, DE), lambda i: (i, 0))
  row_e2 = pl.BlockSpec((BLK, DE), lambda i: (GRID + i, 0))
  def _full(s):
    return pl.BlockSpec(s, lambda i: (0,) * len(s))

  h0 = pl.pallas_call(
      _embed_body, grid=(GRID,),
      in_specs=[row_d, _full((D, D)), _full((1, D))],
      out_specs=row_d,
      out_shape=jax.ShapeDtypeStruct((NP, D), jnp.float32),
  )(x_pad, W0, b0.reshape(1, D))

  (attr_acc,) = _sc_attr_kernel()(edge_attr.reshape(E * DE), dst, z_d, wide)
  (sp0,) = _sc_segsum_kernel()(h0, src, dst, z_d)

  def mlp(sp, h, We, be, W1, b1, W2, b2):
    bemat = (be / DE).reshape(D, 1) * jnp.ones((1, DE), jnp.float32)
    return pl.pallas_call(
        _mlp_body, grid=(GRID,),
        in_specs=[row_d, row_d2, row_d, row_d, row_d2,
                  _full((D, DE)), _full((1, D)), _full((D, DE)),
                  _full((2 * D, D)), _full((1, 2 * D)),
                  _full((D, 2 * D)), _full((1, D))],
        out_specs=row_d,
        out_shape=jax.ShapeDtypeStruct((NP, D), jnp.float32),
    )(sp, sp, h, attr_acc, attr_acc, We, be.reshape(1, D), bemat,
      W1, b1.reshape(1, 2 * D), W2, b2.reshape(1, D))

  def bn(last, x_n, g, bt, out_shape):
    return pl.pallas_call(
        functools.partial(_bn_body, last),
        out_shape=jax.ShapeDtypeStruct(out_shape, jnp.float32),
    )(x_n, g.reshape(1, D), bt.reshape(1, D), batch2, wp_pad, bp_pad)

  out0 = mlp(sp0, h0, We0, be0, W10, b10, W20, b20)
  h1 = bn(False, out0, g0, bt0, (NP, D))

  (sp1,) = _sc_segsum_kernel()(h1, src, dst, z_d)

  out1 = mlp(sp1, h1, We1, be1, W11, b11, W21, b21)
  res = bn(True, out1, g1, bt1, (NG, D))
  return res[:, :1]


# speedup vs baseline: 1.0115x; 1.0115x over previous
import functools

import jax
import jax.numpy as jnp
from jax import lax
from jax.experimental import pallas as pl
from jax.experimental.pallas import tpu as pltpu
from jax.experimental.pallas import tpu_sc as plsc

N = 10000
E = 320000
D = 128
DE = 16
NG = 64

NC = 2
NS = 16
NW = NC * NS
EW = E // NW
C = 80
NCHUNK = EW // C
NP = 10240
RT = NP // NS


def _sc_mesh():
  return plsc.VectorSubcoreMesh(
      core_axis_name="c", subcore_axis_name="s",
      num_cores=NC, num_subcores=NS)


def _sc_segsum_kernel():
  CH = C // 2

  def body(h_hbm, src_hbm, dst_hbm, z_d_hbm, s_out,
           src_v, dst_v, rows0_v, rows1_v, s_sh, gsem, ssem0, ssem1):
    cid = lax.axis_index("c")
    sid = lax.axis_index("s")
    e0 = pl.multiple_of((cid * NS + sid) * EW, 8)
    r0 = pl.multiple_of(sid * RT, 8)

    pltpu.sync_copy(src_hbm.at[pl.ds(e0, EW)], src_v)
    pltpu.sync_copy(dst_hbm.at[pl.ds(e0, EW)], dst_v)
    pltpu.sync_copy(z_d_hbm, s_sh.at[pl.ds(r0, RT)])
    plsc.subcore_barrier()

    def gather_start(i, rows):
      pltpu.async_copy(
          h_hbm.at[src_v.at[pl.ds(i * C, CH)]], rows.at[pl.ds(0, CH)], gsem)
      pltpu.async_copy(
          h_hbm.at[src_v.at[pl.ds(i * C + CH, CH)]],
          rows.at[pl.ds(CH, CH)], gsem)

    def gather_wait(rows):
      pltpu.make_async_copy(h_hbm.at[pl.ds(0, C)], rows, gsem).wait()

    def scatter_start(i, rows, sem):
      pltpu.async_copy(rows, s_sh.at[dst_v.at[pl.ds(i * C, C)]], sem,
                       add=True)

    def scatter_wait(rows, sem):
      pltpu.make_async_copy(h_hbm.at[pl.ds(0, C)], rows, sem).wait()

    gather_start(0, rows0_v)
    gather_wait(rows0_v)
    scatter_start(0, rows0_v, ssem0)
    gather_start(1, rows1_v)
    gather_wait(rows1_v)
    scatter_start(1, rows1_v, ssem1)
    scatter_wait(rows0_v, ssem0)
    gather_start(2, rows0_v)

    def pair(j, carry):
      a = 2 * j
      gather_wait(rows0_v)
      scatter_start(a, rows0_v, ssem0)
      scatter_wait(rows1_v, ssem1)
      gather_start(a + 1, rows1_v)
      gather_wait(rows1_v)
      scatter_start(a + 1, rows1_v, ssem1)
      scatter_wait(rows0_v, ssem0)
      gather_start(a + 2, rows0_v)
      return carry

    lax.fori_loop(1, (NCHUNK - 1) // 2, pair, 0)
    gather_wait(rows0_v)
    scatter_start(NCHUNK - 1, rows0_v, ssem0)
    scatter_wait(rows1_v, ssem1)
    scatter_wait(rows0_v, ssem0)
    plsc.subcore_barrier()

    pltpu.sync_copy(s_sh.at[pl.ds(r0, RT)], s_out.at[pl.ds(cid * NP + r0, RT)])

  return pl.kernel(
      body,
      out_type=[jax.ShapeDtypeStruct((NC * NP, D), jnp.float32)],
      mesh=_sc_mesh(),
      scratch_types=[
          pltpu.VMEM((EW,), jnp.int32),
          pltpu.VMEM((EW,), jnp.int32),
          pltpu.VMEM((C, D), jnp.float32),
          pltpu.VMEM((C, D), jnp.float32),
          pltpu.VMEM_SHARED((NP, D), jnp.float32),
          pltpu.SemaphoreType.DMA,
          pltpu.SemaphoreType.DMA,
          pltpu.SemaphoreType.DMA,
      ])


def _sc_attr_kernel():
  def body(attrf_hbm, dst_hbm, z_d_hbm, wide_hbm, out,
           dst_v, flat0_v, flat1_v, wide0_v, wide1_v, acc_sh,
           gsem, ssem0, ssem1):
    cid = lax.axis_index("c")
    sid = lax.axis_index("s")
    e0 = pl.multiple_of((cid * NS + sid) * EW, 8)
    r0 = pl.multiple_of(sid * RT, 8)

    pltpu.sync_copy(dst_hbm.at[pl.ds(e0, EW)], dst_v)
    pltpu.sync_copy(z_d_hbm, acc_sh.at[pl.ds(r0, RT)])
    pltpu.sync_copy(wide_hbm, wide0_v)
    pltpu.sync_copy(wide_hbm, wide1_v)
    plsc.subcore_barrier()

    def load_start(i, flat):
      pltpu.async_copy(
          attrf_hbm.at[pl.ds((e0 + i * C) * DE, C * DE)], flat, gsem)

    def load_wait(flat):
      pltpu.make_async_copy(attrf_hbm.at[pl.ds(0, C * DE)], flat, gsem).wait()

    def fill(flat, wide):
      for r in range(C):
        wide[r, 0:DE] = flat[pl.ds(r * DE, DE)]

    def scatter_start(i, wide, sem):
      pltpu.async_copy(wide, acc_sh.at[dst_v.at[pl.ds(i * C, C)]], sem,
                       add=True)

    def scatter_wait(wide, sem):
      pltpu.make_async_copy(z_d_hbm.at[pl.ds(0, C)], wide, sem).wait()

    load_start(0, flat0_v)
    load_wait(flat0_v)
    load_start(1, flat1_v)
    fill(flat0_v, wide0_v)
    scatter_start(0, wide0_v, ssem0)
    load_wait(flat1_v)
    load_start(2, flat0_v)
    fill(flat1_v, wide1_v)
    scatter_start(1, wide1_v, ssem1)

    def pair(j, carry):
      a = 2 * j
      load_wait(flat0_v)
      load_start(a + 1, flat1_v)
      scatter_wait(wide0_v, ssem0)
      fill(flat0_v, wide0_v)
      scatter_start(a, wide0_v, ssem0)
      load_wait(flat1_v)
      load_start(a + 2, flat0_v)
      scatter_wait(wide1_v, ssem1)
      fill(flat1_v, wide1_v)
      scatter_start(a + 1, wide1_v, ssem1)
      return carry

    lax.fori_loop(1, (NCHUNK - 1) // 2, pair, 0)
    load_wait(flat0_v)
    scatter_wait(wide0_v, ssem0)
    fill(flat0_v, wide0_v)
    scatter_start(NCHUNK - 1, wide0_v, ssem0)
    scatter_wait(wide1_v, ssem1)
    scatter_wait(wide0_v, ssem0)
    plsc.subcore_barrier()

    pltpu.sync_copy(acc_sh.at[pl.ds(r0, RT)], out.at[pl.ds(cid * NP + r0, RT)])

  return pl.kernel(
      body,
      out_type=[jax.ShapeDtypeStruct((NC * NP, D), jnp.float32)],
      mesh=_sc_mesh(),
      scratch_types=[
          pltpu.VMEM((EW,), jnp.int32),
          pltpu.VMEM((C * DE,), jnp.float32),
          pltpu.VMEM((C * DE,), jnp.float32),
          pltpu.VMEM((C, D), jnp.float32),
          pltpu.VMEM((C, D), jnp.float32),
          pltpu.VMEM_SHARED((NP, D), jnp.float32),
          pltpu.SemaphoreType.DMA,
          pltpu.SemaphoreType.DMA,
          pltpu.SemaphoreType.DMA,
      ])


BLK = 1024
GRID = NP // BLK


def _dot_t(x, w):
  return lax.dot_general(x, w, (((1,), (1,)), ((), ())),
                         preferred_element_type=jnp.float32,
                         precision=lax.Precision.HIGHEST)


def _embed_body(x_ref, w_ref, b_ref, out_ref):
  out_ref[...] = _dot_t(x_ref[...], w_ref[...]) + b_ref[...]


def _mlp_body(sp0_ref, sp1_ref, h_ref, acc0_ref, acc1_ref,
              we_ref, be_ref, bemat_ref, w1_ref, b1_ref, w2_ref, b2_ref,
              out_ref):
  acc = acc0_ref[...] + acc1_ref[...]
  agg = (sp0_ref[...] + sp1_ref[...] + h_ref[...]
         + _dot_t(acc[:, 0:DE], we_ref[...])
         + _dot_t(acc[:, DE:2 * DE], bemat_ref[...])
         + be_ref[...])
  hid = jnp.maximum(_dot_t(agg, w1_ref[...]) + b1_ref[...], 0.0)
  out_ref[...] = _dot_t(hid, w2_ref[...]) + b2_ref[...]


def _bn_body(last, x_ref, g_ref, bt_ref, batch_ref, wp_ref, bp_ref, out_ref):
  x = x_ref[0:N, :]
  mu = jnp.mean(x, axis=0, keepdims=True)
  ctr = x - mu
  var = jnp.mean(ctr * ctr, axis=0, keepdims=True)
  hn = ctr * lax.rsqrt(var + 1e-5) * g_ref[...] + bt_ref[...]
  if not last:
    out_ref[0:N, :] = jnp.maximum(hn, 0.0)
    out_ref[N:NP, :] = jnp.zeros((NP - N, D), jnp.float32)
  else:
    onehot = (batch_ref[...] ==
              lax.broadcasted_iota(jnp.int32, (N, NG), 1)).astype(jnp.float32)
    cnt = jnp.sum(onehot, axis=0, keepdims=True)
    ohs = onehot / jnp.maximum(cnt, 1.0)
    gmean = lax.dot_general(ohs, hn, (((0,), (0,)), ((), ())),
                            preferred_element_type=jnp.float32,
                            precision=lax.Precision.HIGHEST)
    out_ref[...] = _dot_t(gmean, wp_ref[...]) + bp_ref[...]


def kernel(x, edge_index, edge_attr, batch, W0, b0, We0, be0, W10, b10, W20,
           b20, g0, bt0, We1, be1, W11, b11, W21, b21, g1, bt1, Wp, bp):
  assert x.shape == (N, D) and edge_index.shape == (2, E)

  src = edge_index[0]
  dst = edge_index[1]
  z_d = jnp.zeros((RT, D), jnp.float32)
  wide = jnp.zeros((C, D), jnp.float32).at[:, DE:2 * DE].set(1.0)
  batch2 = jnp.broadcast_to(batch[:, None], (N, NG))
  wp_pad = jnp.zeros((D, D), jnp.float32).at[:1, :].set(Wp)
  bp_pad = jnp.zeros((1, D), jnp.float32).at[:, :1].set(bp[None, :])

  row_d = pl.BlockSpec((BLK, D), lambda i: (i, 0))
  row_d2 = pl.BlockSpec((BLK, D), lambda i: (GRID + i, 0))
  row_e = pl.BlockSpec((BLK, DE), lambda i: (i, 0))
  row_e2 = pl.BlockSpec((BLK, DE), lambda i: (GRID + i, 0))
  def _full(s):
    return pl.BlockSpec(s, lambda i: (0,) * len(s))

  h0 = pl.pallas_call(
      _embed_body, grid=(GRID,),
      in_specs=[row_d, _full((D, D)), _full((1, D))],
      out_specs=row_d,
      out_shape=jax.ShapeDtypeStruct((NP, D), jnp.float32),
  )(x, W0, b0.reshape(1, D))

  (attr_acc,) = _sc_attr_kernel()(edge_attr.reshape(E * DE), dst, z_d, wide)
  (sp0,) = _sc_segsum_kernel()(h0, src, dst, z_d)

  def mlp(sp, h, We, be, W1, b1, W2, b2):
    bemat = (be / DE).reshape(D, 1) * jnp.ones((1, DE), jnp.float32)
    return pl.pallas_call(
        _mlp_body, grid=(GRID,),
        in_specs=[row_d, row_d2, row_d, row_d, row_d2,
                  _full((D, DE)), _full((1, D)), _full((D, DE)),
                  _full((2 * D, D)), _full((1, 2 * D)),
                  _full((D, 2 * D)), _full((1, D))],
        out_specs=row_d,
        out_shape=jax.ShapeDtypeStruct((NP, D), jnp.float32),
    )(sp, sp, h, attr_acc, attr_acc, We, be.reshape(1, D), bemat,
      W1, b1.reshape(1, 2 * D), W2, b2.reshape(1, D))

  def bn(last, x_n, g, bt, out_shape):
    return pl.pallas_call(
        functools.partial(_bn_body, last),
        out_shape=jax.ShapeDtypeStruct(out_shape, jnp.float32),
    )(x_n, g.reshape(1, D), bt.reshape(1, D), batch2, wp_pad, bp_pad)

  out0 = mlp(sp0, h0, We0, be0, W10, b10, W20, b20)
  h1 = bn(False, out0, g0, bt0, (NP, D))

  (sp1,) = _sc_segsum_kernel()(h1, src, dst, z_d)

  out1 = mlp(sp1, h1, We1, be1, W11, b11, W21, b21)
  res = bn(True, out1, g1, bt1, (NG, D))
  return res[:, :1]


# speedup vs baseline: 1.0119x; 1.0004x over previous
import functools

import jax
import jax.numpy as jnp
from jax import lax
from jax.experimental import pallas as pl
from jax.experimental.pallas import tpu as pltpu
from jax.experimental.pallas import tpu_sc as plsc

N = 10000
E = 320000
D = 128
DE = 16
NG = 64

NC = 2
NS = 16
NW = NC * NS
EW = E // NW
C = 80
NCHUNK = EW // C
NP = 10240
RT = NP // NS


def _sc_mesh():
  return plsc.VectorSubcoreMesh(
      core_axis_name="c", subcore_axis_name="s",
      num_cores=NC, num_subcores=NS)


def _sc_segsum_kernel():
  def body(h_hbm, src_hbm, dst_hbm, z_d_hbm, s_out,
           src_v, dst_v, rows0_v, rows1_v, s_sh, gsem, ssem0, ssem1):
    cid = lax.axis_index("c")
    sid = lax.axis_index("s")
    e0 = pl.multiple_of((cid * NS + sid) * EW, 8)
    r0 = pl.multiple_of(sid * RT, 8)

    pltpu.sync_copy(src_hbm.at[pl.ds(e0, EW)], src_v)
    pltpu.sync_copy(dst_hbm.at[pl.ds(e0, EW)], dst_v)
    pltpu.sync_copy(z_d_hbm, s_sh.at[pl.ds(r0, RT)])
    plsc.subcore_barrier()

    def gather_start(i, rows):
      pltpu.async_copy(
          h_hbm.at[src_v.at[pl.ds(i * C, C)]], rows, gsem)

    def gather_wait(rows):
      pltpu.make_async_copy(h_hbm.at[pl.ds(0, C)], rows, gsem).wait()

    def scatter_start(i, rows, sem):
      pltpu.async_copy(rows, s_sh.at[dst_v.at[pl.ds(i * C, C)]], sem,
                       add=True)

    def scatter_wait(rows, sem):
      pltpu.make_async_copy(h_hbm.at[pl.ds(0, C)], rows, sem).wait()

    gather_start(0, rows0_v)
    gather_wait(rows0_v)
    scatter_start(0, rows0_v, ssem0)
    gather_start(1, rows1_v)
    gather_wait(rows1_v)
    scatter_start(1, rows1_v, ssem1)
    scatter_wait(rows0_v, ssem0)
    gather_start(2, rows0_v)

    def pair(j, carry):
      a = 2 * j
      gather_wait(rows0_v)
      scatter_start(a, rows0_v, ssem0)
      scatter_wait(rows1_v, ssem1)
      gather_start(a + 1, rows1_v)
      gather_wait(rows1_v)
      scatter_start(a + 1, rows1_v, ssem1)
      scatter_wait(rows0_v, ssem0)
      gather_start(a + 2, rows0_v)
      return carry

    lax.fori_loop(1, (NCHUNK - 1) // 2, pair, 0)
    gather_wait(rows0_v)
    scatter_start(NCHUNK - 1, rows0_v, ssem0)
    scatter_wait(rows1_v, ssem1)
    scatter_wait(rows0_v, ssem0)
    plsc.subcore_barrier()

    pltpu.sync_copy(s_sh.at[pl.ds(r0, RT)], s_out.at[pl.ds(cid * NP + r0, RT)])

  return pl.kernel(
      body,
      out_type=[jax.ShapeDtypeStruct((NC * NP, D), jnp.float32)],
      mesh=_sc_mesh(),
      scratch_types=[
          pltpu.VMEM((EW,), jnp.int32),
          pltpu.VMEM((EW,), jnp.int32),
          pltpu.VMEM((C, D), jnp.float32),
          pltpu.VMEM((C, D), jnp.float32),
          pltpu.VMEM_SHARED((NP, D), jnp.float32),
          pltpu.SemaphoreType.DMA,
          pltpu.SemaphoreType.DMA,
          pltpu.SemaphoreType.DMA,
      ])


def _sc_attr_kernel():
  def body(attrf_hbm, dst_hbm, z_d_hbm, wide_hbm, out,
           dst_v, flat0_v, flat1_v, wide0_v, wide1_v, acc_sh,
           gsem, ssem0, ssem1):
    cid = lax.axis_index("c")
    sid = lax.axis_index("s")
    e0 = pl.multiple_of((cid * NS + sid) * EW, 8)
    r0 = pl.multiple_of(sid * RT, 8)

    pltpu.sync_copy(dst_hbm.at[pl.ds(e0, EW)], dst_v)
    pltpu.sync_copy(z_d_hbm, acc_sh.at[pl.ds(r0, RT)])
    pltpu.sync_copy(wide_hbm, wide0_v)
    pltpu.sync_copy(wide_hbm, wide1_v)
    plsc.subcore_barrier()

    def load_start(i, flat):
      pltpu.async_copy(
          attrf_hbm.at[pl.ds((e0 + i * C) * DE, C * DE)], flat, gsem)

    def load_wait(flat):
      pltpu.make_async_copy(attrf_hbm.at[pl.ds(0, C * DE)], flat, gsem).wait()

    def fill(flat, wide):
      for r in range(C):
        wide[r, 0:DE] = flat[pl.ds(r * DE, DE)]

    def scatter_start(i, wide, sem):
      pltpu.async_copy(wide, acc_sh.at[dst_v.at[pl.ds(i * C, C)]], sem,
                       add=True)

    def scatter_wait(wide, sem):
      pltpu.make_async_copy(z_d_hbm.at[pl.ds(0, C)], wide, sem).wait()

    load_start(0, flat0_v)
    load_wait(flat0_v)
    load_start(1, flat1_v)
    fill(flat0_v, wide0_v)
    scatter_start(0, wide0_v, ssem0)
    load_wait(flat1_v)
    load_start(2, flat0_v)
    fill(flat1_v, wide1_v)
    scatter_start(1, wide1_v, ssem1)

    def pair(j, carry):
      a = 2 * j
      load_wait(flat0_v)
      load_start(a + 1, flat1_v)
      scatter_wait(wide0_v, ssem0)
      fill(flat0_v, wide0_v)
      scatter_start(a, wide0_v, ssem0)
      load_wait(flat1_v)
      load_start(a + 2, flat0_v)
      scatter_wait(wide1_v, ssem1)
      fill(flat1_v, wide1_v)
      scatter_start(a + 1, wide1_v, ssem1)
      return carry

    lax.fori_loop(1, (NCHUNK - 1) // 2, pair, 0)
    load_wait(flat0_v)
    scatter_wait(wide0_v, ssem0)
    fill(flat0_v, wide0_v)
    scatter_start(NCHUNK - 1, wide0_v, ssem0)
    scatter_wait(wide1_v, ssem1)
    scatter_wait(wide0_v, ssem0)
    plsc.subcore_barrier()

    pltpu.sync_copy(acc_sh.at[pl.ds(r0, RT)], out.at[pl.ds(cid * NP + r0, RT)])

  return pl.kernel(
      body,
      out_type=[jax.ShapeDtypeStruct((NC * NP, D), jnp.float32)],
      mesh=_sc_mesh(),
      scratch_types=[
          pltpu.VMEM((EW,), jnp.int32),
          pltpu.VMEM((C * DE,), jnp.float32),
          pltpu.VMEM((C * DE,), jnp.float32),
          pltpu.VMEM((C, D), jnp.float32),
          pltpu.VMEM((C, D), jnp.float32),
          pltpu.VMEM_SHARED((NP, D), jnp.float32),
          pltpu.SemaphoreType.DMA,
          pltpu.SemaphoreType.DMA,
          pltpu.SemaphoreType.DMA,
      ])


BLK = 1024
GRID = NP // BLK


def _dot_t(x, w):
  return lax.dot_general(x, w, (((1,), (1,)), ((), ())),
                         preferred_element_type=jnp.float32,
                         precision=lax.Precision.HIGHEST)


def _embed_body(x_ref, w_ref, b_ref, out_ref):
  out_ref[...] = _dot_t(x_ref[...], w_ref[...]) + b_ref[...]


def _mlp_body(sp0_ref, sp1_ref, h_ref, acc0_ref, acc1_ref,
              we_ref, be_ref, bemat_ref, w1_ref, b1_ref, w2_ref, b2_ref,
              out_ref):
  acc = acc0_ref[...] + acc1_ref[...]
  agg = (sp0_ref[...] + sp1_ref[...] + h_ref[...]
         + _dot_t(acc[:, 0:DE], we_ref[...])
         + _dot_t(acc[:, DE:2 * DE], bemat_ref[...])
         + be_ref[...])
  hid = jnp.maximum(_dot_t(agg, w1_ref[...]) + b1_ref[...], 0.0)
  out_ref[...] = _dot_t(hid, w2_ref[...]) + b2_ref[...]


def _bn_body(last, x_ref, g_ref, bt_ref, batch_ref, wp_ref, bp_ref, out_ref):
  x = x_ref[0:N, :]
  mu = jnp.mean(x, axis=0, keepdims=True)
  ctr = x - mu
  var = jnp.mean(ctr * ctr, axis=0, keepdims=True)
  hn = ctr * lax.rsqrt(var + 1e-5) * g_ref[...] + bt_ref[...]
  if not last:
    out_ref[0:N, :] = jnp.maximum(hn, 0.0)
    out_ref[N:NP, :] = jnp.zeros((NP - N, D), jnp.float32)
  else:
    onehot = (batch_ref[...] ==
              lax.broadcasted_iota(jnp.int32, (N, NG), 1)).astype(jnp.float32)
    cnt = jnp.sum(onehot, axis=0, keepdims=True)
    ohs = onehot / jnp.maximum(cnt, 1.0)
    gmean = lax.dot_general(ohs, hn, (((0,), (0,)), ((), ())),
                            preferred_element_type=jnp.float32,
                            precision=lax.Precision.HIGHEST)
    out_ref[...] = _dot_t(gmean, wp_ref[...]) + bp_ref[...]


def kernel(x, edge_index, edge_attr, batch, W0, b0, We0, be0, W10, b10, W20,
           b20, g0, bt0, We1, be1, W11, b11, W21, b21, g1, bt1, Wp, bp):
  assert x.shape == (N, D) and edge_index.shape == (2, E)

  src = edge_index[0]
  dst = edge_index[1]
  z_d = jnp.zeros((RT, D), jnp.float32)
  wide = jnp.zeros((C, D), jnp.float32).at[:, DE:2 * DE].set(1.0)
  batch2 = jnp.broadcast_to(batch[:, None], (N, NG))
  wp_pad = jnp.zeros((D, D), jnp.float32).at[:1, :].set(Wp)
  bp_pad = jnp.zeros((1, D), jnp.float32).at[:, :1].set(bp[None, :])

  row_d = pl.BlockSpec((BLK, D), lambda i: (i, 0))
  row_d2 = pl.BlockSpec((BLK, D), lambda i: (GRID + i, 0))
  row_e = pl.BlockSpec((BLK, DE), lambda i: (i, 0))
  row_e2 = pl.BlockSpec((BLK, DE), lambda i: (GRID + i, 0))
  def _full(s):
    return pl.BlockSpec(s, lambda i: (0,) * len(s))

  h0 = pl.pallas_call(
      _embed_body, grid=(GRID,),
      in_specs=[row_d, _full((D, D)), _full((1, D))],
      out_specs=row_d,
      out_shape=jax.ShapeDtypeStruct((NP, D), jnp.float32),
  )(x, W0, b0.reshape(1, D))

  (attr_acc,) = _sc_attr_kernel()(edge_attr.reshape(E * DE), dst, z_d, wide)
  (sp0,) = _sc_segsum_kernel()(h0, src, dst, z_d)

  def mlp(sp, h, We, be, W1, b1, W2, b2):
    bemat = (be / DE).reshape(D, 1) * jnp.ones((1, DE), jnp.float32)
    return pl.pallas_call(
        _mlp_body, grid=(GRID,),
        in_specs=[row_d, row_d2, row_d, row_d, row_d2,
                  _full((D, DE)), _full((1, D)), _full((D, DE)),
                  _full((2 * D, D)), _full((1, 2 * D)),
                  _full((D, 2 * D)), _full((1, D))],
        out_specs=row_d,
        out_shape=jax.ShapeDtypeStruct((NP, D), jnp.float32),
    )(sp, sp, h, attr_acc, attr_acc, We, be.reshape(1, D), bemat,
      W1, b1.reshape(1, 2 * D), W2, b2.reshape(1, D))

  def bn(last, x_n, g, bt, out_shape):
    return pl.pallas_call(
        functools.partial(_bn_body, last),
        out_shape=jax.ShapeDtypeStruct(out_shape, jnp.float32),
    )(x_n, g.reshape(1, D), bt.reshape(1, D), batch2, wp_pad, bp_pad)

  out0 = mlp(sp0, h0, We0, be0, W10, b10, W20, b20)
  h1 = bn(False, out0, g0, bt0, (NP, D))

  (sp1,) = _sc_segsum_kernel()(h1, src, dst, z_d)

  out1 = mlp(sp1, h1, We1, be1, W11, b11, W21, b21)
  res = bn(True, out1, g1, bt1, (NG, D))
  return res[:, :1]
